# Initial kernel scaffold; baseline (speedup 1.0000x reference)
import jax
import jax.numpy as jnp
from jax.experimental import pallas as pl


def kernel(xc, zc, xt, zt, params):
    raise NotImplementedError("write your pallas kernel here")


# speedup vs baseline: 2.4918x; 2.4918x over previous
import functools

import jax
import jax.numpy as jnp
import numpy as np
from jax.experimental import pallas as pl
from jax.experimental.pallas import tpu as pltpu

M, H, W, DX, D, NT, NH, WS, K, L, FF = 4, 32, 32, 2, 256, 1024, 8, 8, 9, 2, 512
DH = D // NH
HW = H * W
NWR = H // WS
NWC = W // WS
_SCALE = 1.0 / np.sqrt(DH).astype(np.float32)


def _ln(x, g, b):
    mu = jnp.mean(x, axis=-1, keepdims=True)
    xc = x - mu
    var = jnp.mean(xc * xc, axis=-1, keepdims=True)
    return xc / jnp.sqrt(var + 1e-5) * g + b


def _full_spec(a):
    nd = a.ndim
    return pl.BlockSpec(a.shape, lambda *_: (0,) * nd)


def _topk_body(xt_ref, xc_ref, idx_ref):
    xtb = xt_ref[0]
    xcb = xc_ref[0]
    d0 = xtb[:, 0:1] - xcb[0:1, :]
    d1 = xtb[:, 1:2] - xcb[1:2, :]
    d2 = d0 * d0 + d1 * d1
    iota_c = jax.lax.broadcasted_iota(jnp.int32, (NT, HW), 1)
    cols = []
    for _ in range(K):
        mval = jnp.min(d2, axis=1, keepdims=True)
        cand = jnp.where(d2 == mval, iota_c, HW)
        cidx = jnp.min(cand, axis=1, keepdims=True)
        cols.append(cidx)
        d2 = jnp.where(iota_c == cidx, jnp.float32(np.inf), d2)
    idx_ref[0] = jnp.concatenate(cols, axis=1)


def _topk_call(xt, xc_t):
    return pl.pallas_call(
        _topk_body,
        grid=(M,),
        in_specs=[
            pl.BlockSpec((1, NT, DX), lambda m: (m, 0, 0)),
            pl.BlockSpec((1, DX, HW), lambda m: (m, 0, 0)),
        ],
        out_specs=pl.BlockSpec((1, NT, K), lambda m: (m, 0, 0)),
        out_shape=jax.ShapeDtypeStruct((M, NT, K), jnp.int32),
        compiler_params=pltpu.CompilerParams(
            dimension_semantics=("parallel",)),
    )(xt, xc_t)


def _swin_body(z_ref, wqkv_ref, wo_ref, ln1g_ref, ln1b_ref, ln2g_ref,
               ln2b_ref, w1_ref, b1_ref, w2_ref, b2_ref, out_ref):
    zb = z_ref[0]
    z2 = zb.reshape(WS * W, D)
    zn = _ln(z2, ln1g_ref[...], ln1b_ref[...])
    qkv = jnp.dot(zn, wqkv_ref[...], preferred_element_type=jnp.float32)
    qkv3 = qkv.reshape(WS, W, 3 * D)
    owins = []
    for j in range(NWC):
        w_qkv = qkv3[:, j * WS:(j + 1) * WS, :].reshape(WS * WS, 3 * D)
        oheads = []
        for h in range(NH):
            qh = w_qkv[:, h * DH:(h + 1) * DH]
            kh = w_qkv[:, D + h * DH:D + (h + 1) * DH]
            vh = w_qkv[:, 2 * D + h * DH:2 * D + (h + 1) * DH]
            s = jax.lax.dot_general(
                qh, kh, (((1,), (1,)), ((), ())),
                preferred_element_type=jnp.float32) * _SCALE
            mx = jnp.max(s, axis=1, keepdims=True)
            e = jnp.exp(s - mx)
            p = e / jnp.sum(e, axis=1, keepdims=True)
            oheads.append(jnp.dot(p, vh, preferred_element_type=jnp.float32))
        owins.append(jnp.concatenate(oheads, axis=1).reshape(WS, WS, D))
    o3 = jnp.concatenate(owins, axis=1)
    o2 = o3.reshape(WS * W, D)
    z2 = z2 + jnp.dot(o2, wo_ref[...], preferred_element_type=jnp.float32)
    zn2 = _ln(z2, ln2g_ref[...], ln2b_ref[...])
    h1 = jax.nn.gelu(
        jnp.dot(zn2, w1_ref[...], preferred_element_type=jnp.float32)
        + b1_ref[...])
    z2 = z2 + jnp.dot(h1, w2_ref[...], preferred_element_type=jnp.float32) \
        + b2_ref[...]
    out_ref[0] = z2.reshape(WS, W, D)


def _swin_call(z, p, shift):
    if shift:
        z = jnp.roll(z, (-shift, -shift), (1, 2))
    weights = [p['wqkv'], p['wo'], p['ln1_g'].reshape(1, D),
               p['ln1_b'].reshape(1, D), p['ln2_g'].reshape(1, D),
               p['ln2_b'].reshape(1, D), p['w1'], p['b1'].reshape(1, FF),
               p['w2'], p['b2'].reshape(1, D)]
    out = pl.pallas_call(
        _swin_body,
        grid=(M, NWR),
        in_specs=[pl.BlockSpec((1, WS, W, D), lambda m, r: (m, r, 0, 0))]
        + [_full_spec(w) for w in weights],
        out_specs=pl.BlockSpec((1, WS, W, D), lambda m, r: (m, r, 0, 0)),
        out_shape=jax.ShapeDtypeStruct((M, H, W, D), jnp.float32),
        compiler_params=pltpu.CompilerParams(
            dimension_semantics=("parallel", "parallel")),
    )(z, *weights)
    if shift:
        out = jnp.roll(out, (shift, shift), (1, 2))
    return out


def _mhca_body(zt_ref, zc_ref, idx_ref, wq_ref, wkv_ref, wo_ref, lnqg_ref,
               lnqb_ref, lnkg_ref, lnkb_ref, ln2g_ref, ln2b_ref, w1_ref,
               b1_ref, w2_ref, b2_ref, out_ref):
    ztb = zt_ref[0]
    zcb = zc_ref[0]
    idxb = idx_ref[0]
    qn = _ln(ztb, lnqg_ref[...], lnqb_ref[...])
    q = jnp.dot(qn, wq_ref[...], preferred_element_type=jnp.float32)
    kn = _ln(zcb, lnkg_ref[...], lnkb_ref[...])
    kv = jnp.dot(kn, wkv_ref[...], preferred_element_type=jnp.float32)
    iota_c = jax.lax.broadcasted_iota(jnp.int32, (NT, HW), 1)
    mask = idxb[:, 0:1] == iota_c
    for j in range(1, K):
        mask = jnp.logical_or(mask, idxb[:, j:j + 1] == iota_c)
    oheads = []
    for h in range(NH):
        qh = q[:, h * DH:(h + 1) * DH]
        kh = kv[:, h * DH:(h + 1) * DH]
        vh = kv[:, D + h * DH:D + (h + 1) * DH]
        s = jax.lax.dot_general(
            qh, kh, (((1,), (1,)), ((), ())),
            preferred_element_type=jnp.float32) * _SCALE
        s = jnp.where(mask, s, jnp.float32(-1e30))
        mx = jnp.max(s, axis=1, keepdims=True)
        e = jnp.exp(s - mx)
        p = e / jnp.sum(e, axis=1, keepdims=True)
        oheads.append(jnp.dot(p, vh, preferred_element_type=jnp.float32))
    o = jnp.concatenate(oheads, axis=1)
    z = ztb + jnp.dot(o, wo_ref[...], preferred_element_type=jnp.float32)
    zn2 = _ln(z, ln2g_ref[...], ln2b_ref[...])
    h1 = jax.nn.gelu(
        jnp.dot(zn2, w1_ref[...], preferred_element_type=jnp.float32)
        + b1_ref[...])
    z = z + jnp.dot(h1, w2_ref[...], preferred_element_type=jnp.float32) \
        + b2_ref[...]
    out_ref[0] = z


def _mhca_call(zt, zc_flat, idx, p):
    weights = [p['wq'], p['wkv'], p['wo'], p['lnq_g'].reshape(1, D),
               p['lnq_b'].reshape(1, D), p['lnkv_g'].reshape(1, D),
               p['lnkv_b'].reshape(1, D), p['ln2_g'].reshape(1, D),
               p['ln2_b'].reshape(1, D), p['w1'], p['b1'].reshape(1, FF),
               p['w2'], p['b2'].reshape(1, D)]
    return pl.pallas_call(
        _mhca_body,
        grid=(M,),
        in_specs=[
            pl.BlockSpec((1, NT, D), lambda m: (m, 0, 0)),
            pl.BlockSpec((1, HW, D), lambda m: (m, 0, 0)),
            pl.BlockSpec((1, NT, K), lambda m: (m, 0, 0)),
        ] + [_full_spec(w) for w in weights],
        out_specs=pl.BlockSpec((1, NT, D), lambda m: (m, 0, 0)),
        out_shape=jax.ShapeDtypeStruct((M, NT, D), jnp.float32),
        compiler_params=pltpu.CompilerParams(
            dimension_semantics=("parallel",)),
    )(zt, zc_flat, idx, *weights)


def kernel(xc, zc, xt, zt, params):
    xc_t = xc.reshape(M, HW, DX).transpose(0, 2, 1)
    idx = _topk_call(xt, xc_t)
    for layer in params['layers']:
        zc = _swin_call(zc, layer['swin'][0], 0)
        zc = _swin_call(zc, layer['swin'][1], WS // 2)
        zt = _mhca_call(zt, zc.reshape(M, HW, D), idx, layer['mhca'])
    return zt


# speedup vs baseline: 4.0057x; 1.6075x over previous
import functools

import jax
import jax.numpy as jnp
import numpy as np
from jax.experimental import pallas as pl
from jax.experimental.pallas import tpu as pltpu

M, H, W, DX, D, NT, NH, WS, K, L, FF = 4, 32, 32, 2, 256, 1024, 8, 8, 9, 2, 512
DH = D // NH
HW = H * W
NWR = H // WS
NWC = W // WS
_SCALE = 1.0 / np.sqrt(DH).astype(np.float32)


def _ln(x, g, b):
    mu = jnp.mean(x, axis=-1, keepdims=True)
    xc = x - mu
    var = jnp.mean(xc * xc, axis=-1, keepdims=True)
    return xc / jnp.sqrt(var + 1e-5) * g + b


def _full_spec(a):
    nd = a.ndim
    return pl.BlockSpec(a.shape, lambda *_: (0,) * nd)


def _topk_body(xt_ref, xc_ref, idx_ref):
    xtb = xt_ref[0]
    xcb = xc_ref[0]
    d0 = xtb[:, 0:1] - xcb[0:1, :]
    d1 = xtb[:, 1:2] - xcb[1:2, :]
    d2 = d0 * d0 + d1 * d1
    iota_c = jax.lax.broadcasted_iota(jnp.int32, (NT, HW), 1)
    cols = []
    for _ in range(K):
        mval = jnp.min(d2, axis=1, keepdims=True)
        cand = jnp.where(d2 == mval, iota_c, HW)
        cidx = jnp.min(cand, axis=1, keepdims=True)
        cols.append(cidx)
        d2 = jnp.where(iota_c == cidx, jnp.float32(np.inf), d2)
    idx_ref[0] = jnp.concatenate(cols, axis=1)


def _topk_call(xt, xc_t):
    return pl.pallas_call(
        _topk_body,
        grid=(M,),
        in_specs=[
            pl.BlockSpec((1, NT, DX), lambda m: (m, 0, 0)),
            pl.BlockSpec((1, DX, HW), lambda m: (m, 0, 0)),
        ],
        out_specs=pl.BlockSpec((1, NT, K), lambda m: (m, 0, 0)),
        out_shape=jax.ShapeDtypeStruct((M, NT, K), jnp.int32),
        compiler_params=pltpu.CompilerParams(
            dimension_semantics=("parallel",)),
    )(xt, xc_t)


def _swin_body(z_ref, wqkv_ref, wo_ref, ln1g_ref, ln1b_ref, ln2g_ref,
               ln2b_ref, w1_ref, b1_ref, w2_ref, b2_ref, out_ref):
    zb = z_ref[0]
    TT = WS * W
    z2 = zb.reshape(TT, D)
    zn = _ln(z2, ln1g_ref[...], ln1b_ref[...])
    qkv = jnp.dot(zn, wqkv_ref[...], preferred_element_type=jnp.float32)
    wi = (jax.lax.broadcasted_iota(jnp.int32, (TT, TT), 0) % W) // WS
    wj = (jax.lax.broadcasted_iota(jnp.int32, (TT, TT), 1) % W) // WS
    mask = wi == wj
    oheads = []
    for h in range(NH):
        qh = qkv[:, h * DH:(h + 1) * DH]
        kh = qkv[:, D + h * DH:D + (h + 1) * DH]
        vh = qkv[:, 2 * D + h * DH:2 * D + (h + 1) * DH]
        s = jax.lax.dot_general(
            qh, kh, (((1,), (1,)), ((), ())),
            preferred_element_type=jnp.float32) * _SCALE
        s = jnp.where(mask, s, jnp.float32(-1e30))
        mx = jnp.max(s, axis=1, keepdims=True)
        e = jnp.exp(s - mx)
        p = e / jnp.sum(e, axis=1, keepdims=True)
        oheads.append(jnp.dot(p, vh, preferred_element_type=jnp.float32))
    o2 = jnp.concatenate(oheads, axis=1)
    z2 = z2 + jnp.dot(o2, wo_ref[...], preferred_element_type=jnp.float32)
    zn2 = _ln(z2, ln2g_ref[...], ln2b_ref[...])
    h1 = jax.nn.gelu(
        jnp.dot(zn2, w1_ref[...], preferred_element_type=jnp.float32)
        + b1_ref[...])
    z2 = z2 + jnp.dot(h1, w2_ref[...], preferred_element_type=jnp.float32) \
        + b2_ref[...]
    out_ref[0] = z2.reshape(WS, W, D)


def _swin_call(z, p, shift):
    if shift:
        z = jnp.roll(z, (-shift, -shift), (1, 2))
    weights = [p['wqkv'], p['wo'], p['ln1_g'].reshape(1, D),
               p['ln1_b'].reshape(1, D), p['ln2_g'].reshape(1, D),
               p['ln2_b'].reshape(1, D), p['w1'], p['b1'].reshape(1, FF),
               p['w2'], p['b2'].reshape(1, D)]
    out = pl.pallas_call(
        _swin_body,
        grid=(M, NWR),
        in_specs=[pl.BlockSpec((1, WS, W, D), lambda m, r: (m, r, 0, 0))]
        + [_full_spec(w) for w in weights],
        out_specs=pl.BlockSpec((1, WS, W, D), lambda m, r: (m, r, 0, 0)),
        out_shape=jax.ShapeDtypeStruct((M, H, W, D), jnp.float32),
        compiler_params=pltpu.CompilerParams(
            dimension_semantics=("parallel", "parallel")),
    )(z, *weights)
    if shift:
        out = jnp.roll(out, (shift, shift), (1, 2))
    return out


def _mhca_body(zt_ref, zc_ref, idx_ref, wq_ref, wkv_ref, wo_ref, lnqg_ref,
               lnqb_ref, lnkg_ref, lnkb_ref, ln2g_ref, ln2b_ref, w1_ref,
               b1_ref, w2_ref, b2_ref, out_ref):
    ztb = zt_ref[0]
    zcb = zc_ref[0]
    idxb = idx_ref[0]
    qn = _ln(ztb, lnqg_ref[...], lnqb_ref[...])
    q = jnp.dot(qn, wq_ref[...], preferred_element_type=jnp.float32)
    kn = _ln(zcb, lnkg_ref[...], lnkb_ref[...])
    kv = jnp.dot(kn, wkv_ref[...], preferred_element_type=jnp.float32)
    iota_c = jax.lax.broadcasted_iota(jnp.int32, (NT, HW), 1)
    mask = idxb[:, 0:1] == iota_c
    for j in range(1, K):
        mask = jnp.logical_or(mask, idxb[:, j:j + 1] == iota_c)
    oheads = []
    for h in range(NH):
        qh = q[:, h * DH:(h + 1) * DH]
        kh = kv[:, h * DH:(h + 1) * DH]
        vh = kv[:, D + h * DH:D + (h + 1) * DH]
        s = jax.lax.dot_general(
            qh, kh, (((1,), (1,)), ((), ())),
            preferred_element_type=jnp.float32) * _SCALE
        s = jnp.where(mask, s, jnp.float32(-1e30))
        mx = jnp.max(s, axis=1, keepdims=True)
        e = jnp.exp(s - mx)
        p = e / jnp.sum(e, axis=1, keepdims=True)
        oheads.append(jnp.dot(p, vh, preferred_element_type=jnp.float32))
    o = jnp.concatenate(oheads, axis=1)
    z = ztb + jnp.dot(o, wo_ref[...], preferred_element_type=jnp.float32)
    zn2 = _ln(z, ln2g_ref[...], ln2b_ref[...])
    h1 = jax.nn.gelu(
        jnp.dot(zn2, w1_ref[...], preferred_element_type=jnp.float32)
        + b1_ref[...])
    z = z + jnp.dot(h1, w2_ref[...], preferred_element_type=jnp.float32) \
        + b2_ref[...]
    out_ref[0] = z


def _mhca_call(zt, zc_flat, idx, p):
    weights = [p['wq'], p['wkv'], p['wo'], p['lnq_g'].reshape(1, D),
               p['lnq_b'].reshape(1, D), p['lnkv_g'].reshape(1, D),
               p['lnkv_b'].reshape(1, D), p['ln2_g'].reshape(1, D),
               p['ln2_b'].reshape(1, D), p['w1'], p['b1'].reshape(1, FF),
               p['w2'], p['b2'].reshape(1, D)]
    return pl.pallas_call(
        _mhca_body,
        grid=(M,),
        in_specs=[
            pl.BlockSpec((1, NT, D), lambda m: (m, 0, 0)),
            pl.BlockSpec((1, HW, D), lambda m: (m, 0, 0)),
            pl.BlockSpec((1, NT, K), lambda m: (m, 0, 0)),
        ] + [_full_spec(w) for w in weights],
        out_specs=pl.BlockSpec((1, NT, D), lambda m: (m, 0, 0)),
        out_shape=jax.ShapeDtypeStruct((M, NT, D), jnp.float32),
        compiler_params=pltpu.CompilerParams(
            dimension_semantics=("parallel",)),
    )(zt, zc_flat, idx, *weights)


def kernel(xc, zc, xt, zt, params):
    xc_t = xc.reshape(M, HW, DX).transpose(0, 2, 1)
    idx = _topk_call(xt, xc_t)
    for layer in params['layers']:
        zc = _swin_call(zc, layer['swin'][0], 0)
        zc = _swin_call(zc, layer['swin'][1], WS // 2)
        zt = _mhca_call(zt, zc.reshape(M, HW, D), idx, layer['mhca'])
    return zt


# speedup vs baseline: 4.9804x; 1.2433x over previous
import functools

import jax
import jax.numpy as jnp
import numpy as np
from jax import lax
from jax.experimental import pallas as pl
from jax.experimental.pallas import tpu as pltpu
from jax.experimental.pallas import tpu_sc as plsc

M, H, W, DX, D, NT, NH, WS, K, L, FF = 4, 32, 32, 2, 256, 1024, 8, 8, 9, 2, 512
DH = D // NH
HW = H * W
NWR = H // WS
NWC = W // WS
_SCALE = 1.0 / np.sqrt(DH).astype(np.float32)


def _ln(x, g, b):
    mu = jnp.mean(x, axis=-1, keepdims=True)
    xc = x - mu
    var = jnp.mean(xc * xc, axis=-1, keepdims=True)
    return xc / jnp.sqrt(var + 1e-5) * g + b


def _full_spec(a):
    nd = a.ndim
    return pl.BlockSpec(a.shape, lambda *_: (0,) * nd)


def _topk_body(xt_ref, xc_ref, idx_ref):
    xtb = xt_ref[0]
    xcb = xc_ref[0]
    d0 = xtb[:, 0:1] - xcb[0:1, :]
    d1 = xtb[:, 1:2] - xcb[1:2, :]
    d2 = d0 * d0 + d1 * d1
    iota_c = jax.lax.broadcasted_iota(jnp.int32, (NT, HW), 1)
    cols = []
    for _ in range(K):
        mval = jnp.min(d2, axis=1, keepdims=True)
        cand = jnp.where(d2 == mval, iota_c, HW)
        cidx = jnp.min(cand, axis=1, keepdims=True)
        cols.append(cidx)
        d2 = jnp.where(iota_c == cidx, jnp.float32(np.inf), d2)
    idx_ref[0] = jnp.concatenate(cols, axis=1)


def _topk_call(xt, xc_t):
    return pl.pallas_call(
        _topk_body,
        grid=(M,),
        in_specs=[
            pl.BlockSpec((1, NT, DX), lambda m: (m, 0, 0)),
            pl.BlockSpec((1, DX, HW), lambda m: (m, 0, 0)),
        ],
        out_specs=pl.BlockSpec((1, NT, K), lambda m: (m, 0, 0)),
        out_shape=jax.ShapeDtypeStruct((M, NT, K), jnp.int32),
        compiler_params=pltpu.CompilerParams(
            dimension_semantics=("parallel",)),
    )(xt, xc_t)


def _swin_body(z_ref, wqkv_ref, wo_ref, ln1g_ref, ln1b_ref, ln2g_ref,
               ln2b_ref, w1_ref, b1_ref, w2_ref, b2_ref, out_ref):
    zb = z_ref[0]
    TT = WS * W
    z2 = zb.reshape(TT, D)
    zn = _ln(z2, ln1g_ref[...], ln1b_ref[...])
    qkv = jnp.dot(zn, wqkv_ref[...], preferred_element_type=jnp.float32)
    wi = (jax.lax.broadcasted_iota(jnp.int32, (TT, TT), 0) % W) // WS
    wj = (jax.lax.broadcasted_iota(jnp.int32, (TT, TT), 1) % W) // WS
    mask = wi == wj
    oheads = []
    for h in range(NH):
        qh = qkv[:, h * DH:(h + 1) * DH]
        kh = qkv[:, D + h * DH:D + (h + 1) * DH]
        vh = qkv[:, 2 * D + h * DH:2 * D + (h + 1) * DH]
        s = jax.lax.dot_general(
            qh, kh, (((1,), (1,)), ((), ())),
            preferred_element_type=jnp.float32) * _SCALE
        s = jnp.where(mask, s, jnp.float32(-1e30))
        mx = jnp.max(s, axis=1, keepdims=True)
        e = jnp.exp(s - mx)
        p = e / jnp.sum(e, axis=1, keepdims=True)
        oheads.append(jnp.dot(p, vh, preferred_element_type=jnp.float32))
    o2 = jnp.concatenate(oheads, axis=1)
    z2 = z2 + jnp.dot(o2, wo_ref[...], preferred_element_type=jnp.float32)
    zn2 = _ln(z2, ln2g_ref[...], ln2b_ref[...])
    h1 = jax.nn.gelu(
        jnp.dot(zn2, w1_ref[...], preferred_element_type=jnp.float32)
        + b1_ref[...])
    z2 = z2 + jnp.dot(h1, w2_ref[...], preferred_element_type=jnp.float32) \
        + b2_ref[...]
    out_ref[0] = z2.reshape(WS, W, D)


def _swin_call(z, p, shift):
    if shift:
        z = jnp.roll(z, (-shift, -shift), (1, 2))
    weights = [p['wqkv'], p['wo'], p['ln1_g'].reshape(1, D),
               p['ln1_b'].reshape(1, D), p['ln2_g'].reshape(1, D),
               p['ln2_b'].reshape(1, D), p['w1'], p['b1'].reshape(1, FF),
               p['w2'], p['b2'].reshape(1, D)]
    out = pl.pallas_call(
        _swin_body,
        grid=(M, NWR),
        in_specs=[pl.BlockSpec((1, WS, W, D), lambda m, r: (m, r, 0, 0))]
        + [_full_spec(w) for w in weights],
        out_specs=pl.BlockSpec((1, WS, W, D), lambda m, r: (m, r, 0, 0)),
        out_shape=jax.ShapeDtypeStruct((M, H, W, D), jnp.float32),
        compiler_params=pltpu.CompilerParams(
            dimension_semantics=("parallel", "parallel")),
    )(z, *weights)
    if shift:
        out = jnp.roll(out, (shift, shift), (1, 2))
    return out


def _kvproj_body(zc_ref, wkv_ref, g_ref, b_ref, out_ref):
    kn = _ln(zc_ref[0], g_ref[...], b_ref[...])
    out_ref[0] = jnp.dot(kn, wkv_ref[...], preferred_element_type=jnp.float32)


def _kvproj_call(zc_flat, p):
    weights = [p['wkv'], p['lnkv_g'].reshape(1, D), p['lnkv_b'].reshape(1, D)]
    return pl.pallas_call(
        _kvproj_body,
        grid=(M,),
        in_specs=[pl.BlockSpec((1, HW, D), lambda m: (m, 0, 0))]
        + [_full_spec(w) for w in weights],
        out_specs=pl.BlockSpec((1, HW, 2 * D), lambda m: (m, 0, 0)),
        out_shape=jax.ShapeDtypeStruct((M, HW, 2 * D), jnp.float32),
        compiler_params=pltpu.CompilerParams(
            dimension_semantics=("parallel",)),
    )(zc_flat, *weights)


_GB = K * M * NT
_CH = 128


@functools.cache
def _make_sc_gather():
    info = plsc.get_sparse_core_info()
    nw = info.num_cores * info.num_subcores
    b_per_w = _GB // nw
    n_chunks = b_per_w // _CH
    mesh = plsc.VectorSubcoreMesh(core_axis_name="c", subcore_axis_name="s")

    @functools.partial(
        pl.kernel, mesh=mesh,
        out_type=jax.ShapeDtypeStruct((_GB, 2 * D), jnp.float32),
        scratch_types=[
            pltpu.VMEM((_CH,), jnp.int32),
            pltpu.VMEM((_CH, 2 * D), jnp.float32),
            pltpu.SemaphoreType.DMA,
        ],
    )
    def sc_gather(table_hbm, idx_hbm, out_hbm, idx_v, rows_v, sem):
        wid = lax.axis_index("s") * info.num_cores + lax.axis_index("c")
        base = wid * b_per_w
        for c in range(n_chunks):
            off = base + c * _CH
            pltpu.sync_copy(idx_hbm.at[pl.ds(off, _CH)], idx_v)
            pltpu.async_copy(table_hbm.at[idx_v], rows_v, sem).wait()
            pltpu.sync_copy(rows_v, out_hbm.at[pl.ds(off, _CH)])

    return sc_gather


def _gather_rows(table, idx_flat):
    return _make_sc_gather()(table, idx_flat)


def _mhca_body(zt_ref, g_ref, wq_ref, wo_ref, lnqg_ref, lnqb_ref, ln2g_ref,
               ln2b_ref, w1_ref, b1_ref, w2_ref, b2_ref, out_ref):
    ztb = zt_ref[0]
    qn = _ln(ztb, lnqg_ref[...], lnqb_ref[...])
    q = jnp.dot(qn, wq_ref[...], preferred_element_type=jnp.float32)
    seg = (jax.lax.broadcasted_iota(jnp.int32, (D, NH), 0) // DH
           == jax.lax.broadcasted_iota(jnp.int32, (D, NH), 1)
           ).astype(jnp.float32)
    segT = (jax.lax.broadcasted_iota(jnp.int32, (NH, D), 0)
            == jax.lax.broadcasted_iota(jnp.int32, (NH, D), 1) // DH
            ).astype(jnp.float32)
    s_list = []
    for j in range(K):
        kj = g_ref[j, 0, :, 0:D]
        s_list.append(jnp.dot(q * kj, seg,
                              preferred_element_type=jnp.float32) * _SCALE)
    mx = s_list[0]
    for j in range(1, K):
        mx = jnp.maximum(mx, s_list[j])
    e_list = [jnp.exp(s - mx) for s in s_list]
    den = e_list[0]
    for j in range(1, K):
        den = den + e_list[j]
    inv = 1.0 / den
    acc = jnp.zeros((NT, D), jnp.float32)
    for j in range(K):
        vj = g_ref[j, 0, :, D:2 * D]
        a_exp = jnp.dot(e_list[j] * inv, segT,
                        preferred_element_type=jnp.float32)
        acc = acc + a_exp * vj
    z = ztb + jnp.dot(acc, wo_ref[...], preferred_element_type=jnp.float32)
    zn2 = _ln(z, ln2g_ref[...], ln2b_ref[...])
    h1 = jax.nn.gelu(
        jnp.dot(zn2, w1_ref[...], preferred_element_type=jnp.float32)
        + b1_ref[...])
    z = z + jnp.dot(h1, w2_ref[...], preferred_element_type=jnp.float32) \
        + b2_ref[...]
    out_ref[0] = z


def _mhca_call(zt, gathered, p):
    g4 = gathered.reshape(K, M, NT, 2 * D)
    weights = [p['wq'], p['wo'], p['lnq_g'].reshape(1, D),
               p['lnq_b'].reshape(1, D), p['ln2_g'].reshape(1, D),
               p['ln2_b'].reshape(1, D), p['w1'], p['b1'].reshape(1, FF),
               p['w2'], p['b2'].reshape(1, D)]
    return pl.pallas_call(
        _mhca_body,
        grid=(M,),
        in_specs=[
            pl.BlockSpec((1, NT, D), lambda m: (m, 0, 0)),
            pl.BlockSpec((K, 1, NT, 2 * D), lambda m: (0, m, 0, 0)),
        ] + [_full_spec(w) for w in weights],
        out_specs=pl.BlockSpec((1, NT, D), lambda m: (m, 0, 0)),
        out_shape=jax.ShapeDtypeStruct((M, NT, D), jnp.float32),
        compiler_params=pltpu.CompilerParams(
            dimension_semantics=("parallel",)),
    )(zt, g4, *weights)


def kernel(xc, zc, xt, zt, params):
    xc_t = xc.reshape(M, HW, DX).transpose(0, 2, 1)
    idx = _topk_call(xt, xc_t)
    idx_flat = (idx.transpose(2, 0, 1)
                + (HW * jnp.arange(M, dtype=jnp.int32))[None, :, None]
                ).reshape(_GB)
    for layer in params['layers']:
        zc = _swin_call(zc, layer['swin'][0], 0)
        zc = _swin_call(zc, layer['swin'][1], WS // 2)
        kv = _kvproj_call(zc.reshape(M, HW, D), layer['mhca'])
        gathered = _gather_rows(kv.reshape(M * HW, 2 * D), idx_flat)
        zt = _mhca_call(zt, gathered, layer['mhca'])
    return zt


# speedup vs baseline: 5.0596x; 1.0159x over previous
import functools

import jax
import jax.numpy as jnp
import numpy as np
from jax import lax
from jax.experimental import pallas as pl
from jax.experimental.pallas import tpu as pltpu
from jax.experimental.pallas import tpu_sc as plsc

M, H, W, DX, D, NT, NH, WS, K, L, FF = 4, 32, 32, 2, 256, 1024, 8, 8, 9, 2, 512
DH = D // NH
HW = H * W
NWR = H // WS
NWC = W // WS
_SCALE = 1.0 / np.sqrt(DH).astype(np.float32)


def _ln(x, g, b):
    mu = jnp.mean(x, axis=-1, keepdims=True)
    xc = x - mu
    var = jnp.mean(xc * xc, axis=-1, keepdims=True)
    return xc / jnp.sqrt(var + 1e-5) * g + b


def _full_spec(a):
    nd = a.ndim
    return pl.BlockSpec(a.shape, lambda *_: (0,) * nd)


def _topk_body(xt_ref, xc_ref, idx_ref):
    xtb = xt_ref[0]
    xcb = xc_ref[0]
    d0 = xtb[:, 0:1] - xcb[0:1, :]
    d1 = xtb[:, 1:2] - xcb[1:2, :]
    d2 = d0 * d0 + d1 * d1
    iota_c = jax.lax.broadcasted_iota(jnp.int32, (NT, HW), 1)
    cols = []
    for _ in range(K):
        mval = jnp.min(d2, axis=1, keepdims=True)
        cand = jnp.where(d2 == mval, iota_c, HW)
        cidx = jnp.min(cand, axis=1, keepdims=True)
        cols.append(cidx)
        d2 = jnp.where(iota_c == cidx, jnp.float32(np.inf), d2)
    idx_ref[0] = jnp.concatenate(cols, axis=1)


def _topk_call(xt, xc_t):
    return pl.pallas_call(
        _topk_body,
        grid=(M,),
        in_specs=[
            pl.BlockSpec((1, NT, DX), lambda m: (m, 0, 0)),
            pl.BlockSpec((1, DX, HW), lambda m: (m, 0, 0)),
        ],
        out_specs=pl.BlockSpec((1, NT, K), lambda m: (m, 0, 0)),
        out_shape=jax.ShapeDtypeStruct((M, NT, K), jnp.int32),
        compiler_params=pltpu.CompilerParams(
            dimension_semantics=("parallel",)),
    )(xt, xc_t)


def _swin_body(z_ref, wqkv_ref, wo_ref, ln1g_ref, ln1b_ref, ln2g_ref,
               ln2b_ref, w1_ref, b1_ref, w2_ref, b2_ref, out_ref):
    zb = z_ref[0]
    TT = WS * W
    z2 = zb.reshape(TT, D)
    zn = _ln(z2, ln1g_ref[...], ln1b_ref[...])
    qkv = jnp.dot(zn.astype(jnp.bfloat16), wqkv_ref[...],
                  preferred_element_type=jnp.float32)
    wi = (jax.lax.broadcasted_iota(jnp.int32, (TT, TT), 0) % W) // WS
    wj = (jax.lax.broadcasted_iota(jnp.int32, (TT, TT), 1) % W) // WS
    mask = wi == wj
    oheads = []
    for h in range(NH):
        qh = qkv[:, h * DH:(h + 1) * DH]
        kh = qkv[:, D + h * DH:D + (h + 1) * DH]
        vh = qkv[:, 2 * D + h * DH:2 * D + (h + 1) * DH]
        s = jax.lax.dot_general(
            qh, kh, (((1,), (1,)), ((), ())),
            preferred_element_type=jnp.float32) * _SCALE
        s = jnp.where(mask, s, jnp.float32(-1e30))
        mx = jnp.max(s, axis=1, keepdims=True)
        e = jnp.exp(s - mx)
        p = e / jnp.sum(e, axis=1, keepdims=True)
        oheads.append(jnp.dot(p, vh, preferred_element_type=jnp.float32))
    o2 = jnp.concatenate(oheads, axis=1)
    z2 = z2 + jnp.dot(o2.astype(jnp.bfloat16), wo_ref[...],
                      preferred_element_type=jnp.float32)
    zn2 = _ln(z2, ln2g_ref[...], ln2b_ref[...])
    h1 = jax.nn.gelu(
        jnp.dot(zn2.astype(jnp.bfloat16), w1_ref[...],
                preferred_element_type=jnp.float32) + b1_ref[...])
    z2 = z2 + jnp.dot(h1.astype(jnp.bfloat16), w2_ref[...],
                      preferred_element_type=jnp.float32) + b2_ref[...]
    out_ref[0] = z2.reshape(WS, W, D)


def _swin_call(z, p, shift):
    if shift:
        z = jnp.roll(z, (-shift, -shift), (1, 2))
    bf = jnp.bfloat16
    weights = [p['wqkv'].astype(bf), p['wo'].astype(bf),
               p['ln1_g'].reshape(1, D),
               p['ln1_b'].reshape(1, D), p['ln2_g'].reshape(1, D),
               p['ln2_b'].reshape(1, D), p['w1'].astype(bf),
               p['b1'].reshape(1, FF),
               p['w2'].astype(bf), p['b2'].reshape(1, D)]
    out = pl.pallas_call(
        _swin_body,
        grid=(M, NWR),
        in_specs=[pl.BlockSpec((1, WS, W, D), lambda m, r: (m, r, 0, 0))]
        + [_full_spec(w) for w in weights],
        out_specs=pl.BlockSpec((1, WS, W, D), lambda m, r: (m, r, 0, 0)),
        out_shape=jax.ShapeDtypeStruct((M, H, W, D), jnp.float32),
        compiler_params=pltpu.CompilerParams(
            dimension_semantics=("parallel", "parallel")),
    )(z, *weights)
    if shift:
        out = jnp.roll(out, (shift, shift), (1, 2))
    return out


def _kvproj_body(zc_ref, wkv_ref, g_ref, b_ref, out_ref):
    kn = _ln(zc_ref[0], g_ref[...], b_ref[...])
    kv = jnp.dot(kn.astype(jnp.bfloat16), wkv_ref[...],
                 preferred_element_type=jnp.float32)
    kb = jax.lax.bitcast_convert_type(
        kv[:, 0:D].astype(jnp.bfloat16), jnp.uint16).astype(jnp.int32)
    vb = jax.lax.bitcast_convert_type(
        kv[:, D:2 * D].astype(jnp.bfloat16), jnp.uint16).astype(jnp.int32)
    out_ref[0] = jax.lax.shift_left(kb, 16) | vb


def _kvproj_call(zc_flat, p):
    weights = [p['wkv'].astype(jnp.bfloat16), p['lnkv_g'].reshape(1, D),
               p['lnkv_b'].reshape(1, D)]
    return pl.pallas_call(
        _kvproj_body,
        grid=(M,),
        in_specs=[pl.BlockSpec((1, HW, D), lambda m: (m, 0, 0))]
        + [_full_spec(w) for w in weights],
        out_specs=pl.BlockSpec((1, HW, D), lambda m: (m, 0, 0)),
        out_shape=jax.ShapeDtypeStruct((M, HW, D), jnp.int32),
        compiler_params=pltpu.CompilerParams(
            dimension_semantics=("parallel",)),
    )(zc_flat, *weights)


_GB = K * M * NT
_CH = 128


@functools.cache
def _make_sc_gather():
    info = plsc.get_sparse_core_info()
    nw = info.num_cores * info.num_subcores
    b_per_w = _GB // nw
    n_chunks = b_per_w // _CH
    mesh = plsc.VectorSubcoreMesh(core_axis_name="c", subcore_axis_name="s")

    @functools.partial(
        pl.kernel, mesh=mesh,
        out_type=jax.ShapeDtypeStruct((_GB, D), jnp.int32),
        scratch_types=[
            pltpu.VMEM((b_per_w,), jnp.int32),
            pltpu.VMEM((2, _CH, D), jnp.int32),
            pltpu.SemaphoreType.DMA,
            pltpu.SemaphoreType.DMA,
        ],
    )
    def sc_gather(table_hbm, idx_hbm, out_hbm, idx_v, rows_v, sem0, sem1):
        wid = lax.axis_index("s") * info.num_cores + lax.axis_index("c")
        base = wid * b_per_w
        pltpu.sync_copy(idx_hbm.at[pl.ds(base, b_per_w)], idx_v)
        sems = (sem0, sem1)
        cps = {}
        cps[0] = pltpu.async_copy(
            table_hbm.at[idx_v.at[pl.ds(0, _CH)]], rows_v.at[0], sems[0])
        for c in range(1, n_chunks):
            b = c % 2
            cps[c] = pltpu.async_copy(
                table_hbm.at[idx_v.at[pl.ds(c * _CH, _CH)]],
                rows_v.at[b], sems[b])
            cps[c - 1].wait()
            pltpu.sync_copy(rows_v.at[1 - b],
                            out_hbm.at[pl.ds(base + (c - 1) * _CH, _CH)])
        last = n_chunks - 1
        cps[last].wait()
        pltpu.sync_copy(rows_v.at[last % 2],
                        out_hbm.at[pl.ds(base + last * _CH, _CH)])

    return sc_gather


def _gather_rows(table, idx_flat):
    return _make_sc_gather()(table, idx_flat)


def _mhca_body(zt_ref, g_ref, wq_ref, wo_ref, lnqg_ref, lnqb_ref, ln2g_ref,
               ln2b_ref, w1_ref, b1_ref, w2_ref, b2_ref, out_ref):
    ztb = zt_ref[0]
    qn = _ln(ztb, lnqg_ref[...], lnqb_ref[...])
    q = jnp.dot(qn.astype(jnp.bfloat16), wq_ref[...],
                preferred_element_type=jnp.float32)
    seg = (jax.lax.broadcasted_iota(jnp.int32, (D, NH), 0) // DH
           == jax.lax.broadcasted_iota(jnp.int32, (D, NH), 1)
           ).astype(jnp.float32)
    segT = (jax.lax.broadcasted_iota(jnp.int32, (NH, D), 0)
            == jax.lax.broadcasted_iota(jnp.int32, (NH, D), 1) // DH
            ).astype(jnp.float32)
    def _unpack_hi(w):
        return jax.lax.bitcast_convert_type(
            jax.lax.shift_right_logical(w, 16).astype(jnp.uint16),
            jnp.bfloat16).astype(jnp.float32)

    def _unpack_lo(w):
        return jax.lax.bitcast_convert_type(
            w.astype(jnp.uint16), jnp.bfloat16).astype(jnp.float32)

    s_list = []
    for j in range(K):
        kj = _unpack_hi(g_ref[j, 0])
        s_list.append(jnp.dot(q * kj, seg,
                              preferred_element_type=jnp.float32) * _SCALE)
    mx = s_list[0]
    for j in range(1, K):
        mx = jnp.maximum(mx, s_list[j])
    e_list = [jnp.exp(s - mx) for s in s_list]
    den = e_list[0]
    for j in range(1, K):
        den = den + e_list[j]
    inv = 1.0 / den
    acc = jnp.zeros((NT, D), jnp.float32)
    for j in range(K):
        vj = _unpack_lo(g_ref[j, 0])
        a_exp = jnp.dot(e_list[j] * inv, segT,
                        preferred_element_type=jnp.float32)
        acc = acc + a_exp * vj
    z = ztb + jnp.dot(acc.astype(jnp.bfloat16), wo_ref[...],
                      preferred_element_type=jnp.float32)
    zn2 = _ln(z, ln2g_ref[...], ln2b_ref[...])
    h1 = jax.nn.gelu(
        jnp.dot(zn2.astype(jnp.bfloat16), w1_ref[...],
                preferred_element_type=jnp.float32) + b1_ref[...])
    z = z + jnp.dot(h1.astype(jnp.bfloat16), w2_ref[...],
                    preferred_element_type=jnp.float32) + b2_ref[...]
    out_ref[0] = z


def _mhca_call(zt, gathered, p):
    g4 = gathered.reshape(K, M, NT, D)
    bf = jnp.bfloat16
    weights = [p['wq'].astype(bf), p['wo'].astype(bf),
               p['lnq_g'].reshape(1, D),
               p['lnq_b'].reshape(1, D), p['ln2_g'].reshape(1, D),
               p['ln2_b'].reshape(1, D), p['w1'].astype(bf),
               p['b1'].reshape(1, FF),
               p['w2'].astype(bf), p['b2'].reshape(1, D)]
    return pl.pallas_call(
        _mhca_body,
        grid=(M,),
        in_specs=[
            pl.BlockSpec((1, NT, D), lambda m: (m, 0, 0)),
            pl.BlockSpec((K, 1, NT, D), lambda m: (0, m, 0, 0)),
        ] + [_full_spec(w) for w in weights],
        out_specs=pl.BlockSpec((1, NT, D), lambda m: (m, 0, 0)),
        out_shape=jax.ShapeDtypeStruct((M, NT, D), jnp.float32),
        compiler_params=pltpu.CompilerParams(
            dimension_semantics=("parallel",)),
    )(zt, g4, *weights)


def kernel(xc, zc, xt, zt, params):
    xc_t = xc.reshape(M, HW, DX).transpose(0, 2, 1)
    idx = _topk_call(xt, xc_t)
    idx_flat = (idx.transpose(2, 0, 1)
                + (HW * jnp.arange(M, dtype=jnp.int32))[None, :, None]
                ).reshape(_GB)
    for layer in params['layers']:
        zc = _swin_call(zc, layer['swin'][0], 0)
        zc = _swin_call(zc, layer['swin'][1], WS // 2)
        kv = _kvproj_call(zc.reshape(M, HW, D), layer['mhca'])
        gathered = _gather_rows(kv.reshape(M * HW, D), idx_flat)
        zt = _mhca_call(zt, gathered, layer['mhca'])
    return zt


# speedup vs baseline: 5.3971x; 1.0667x over previous
import functools

import jax
import jax.numpy as jnp
import numpy as np
from jax import lax
from jax.experimental import pallas as pl
from jax.experimental.pallas import tpu as pltpu
from jax.experimental.pallas import tpu_sc as plsc

M, H, W, DX, D, NT, NH, WS, K, L, FF = 4, 32, 32, 2, 256, 1024, 8, 8, 9, 2, 512
DH = D // NH
HW = H * W
NWR = H // WS
NWC = W // WS
_SCALE = 1.0 / np.sqrt(DH).astype(np.float32)


def _ln(x, g, b):
    mu = jnp.mean(x, axis=-1, keepdims=True)
    xc = x - mu
    var = jnp.mean(xc * xc, axis=-1, keepdims=True)
    return xc / jnp.sqrt(var + 1e-5) * g + b


def _full_spec(a):
    nd = a.ndim
    return pl.BlockSpec(a.shape, lambda *_: (0,) * nd)


def _topk_body(xt_ref, xc_ref, idx_ref):
    xtb = xt_ref[0]
    xcb = xc_ref[0]
    d0 = xtb[:, 0:1] - xcb[0:1, :]
    d1 = xtb[:, 1:2] - xcb[1:2, :]
    d2 = d0 * d0 + d1 * d1
    iota_c = jax.lax.broadcasted_iota(jnp.int32, (NT, HW), 1)
    cols = []
    for _ in range(K):
        mval = jnp.min(d2, axis=1, keepdims=True)
        cand = jnp.where(d2 == mval, iota_c, HW)
        cidx = jnp.min(cand, axis=1, keepdims=True)
        cols.append(cidx)
        d2 = jnp.where(iota_c == cidx, jnp.float32(np.inf), d2)
    idx_ref[0] = jnp.concatenate(cols, axis=1)


def _topk_call(xt, xc_t):
    return pl.pallas_call(
        _topk_body,
        grid=(M,),
        in_specs=[
            pl.BlockSpec((1, NT, DX), lambda m: (m, 0, 0)),
            pl.BlockSpec((1, DX, HW), lambda m: (m, 0, 0)),
        ],
        out_specs=pl.BlockSpec((1, NT, K), lambda m: (m, 0, 0)),
        out_shape=jax.ShapeDtypeStruct((M, NT, K), jnp.int32),
        compiler_params=pltpu.CompilerParams(
            dimension_semantics=("parallel",)),
    )(xt, xc_t)


def _swin_body(z_ref, wqkv_ref, wo_ref, ln1g_ref, ln1b_ref, ln2g_ref,
               ln2b_ref, w1_ref, b1_ref, w2_ref, b2_ref, out_ref):
    zb = z_ref[0]
    TT = 2 * WS * W
    z2 = zb.reshape(TT, D)
    zn = _ln(z2, ln1g_ref[...], ln1b_ref[...])
    qkv = jnp.dot(zn.astype(jnp.bfloat16), wqkv_ref[...],
                  preferred_element_type=jnp.float32)
    ti = jax.lax.broadcasted_iota(jnp.int32, (TT, TT), 0)
    tj = jax.lax.broadcasted_iota(jnp.int32, (TT, TT), 1)
    wi = (ti // (WS * W)) * NWC + (ti % W) // WS
    wj = (tj // (WS * W)) * NWC + (tj % W) // WS
    mask = wi == wj
    oheads = []
    for h in range(NH):
        qh = qkv[:, h * DH:(h + 1) * DH]
        kh = qkv[:, D + h * DH:D + (h + 1) * DH]
        vh = qkv[:, 2 * D + h * DH:2 * D + (h + 1) * DH]
        s = jax.lax.dot_general(
            qh, kh, (((1,), (1,)), ((), ())),
            preferred_element_type=jnp.float32) * _SCALE
        s = jnp.where(mask, s, jnp.float32(-1e30))
        mx = jnp.max(s, axis=1, keepdims=True)
        e = jnp.exp(s - mx)
        p = e / jnp.sum(e, axis=1, keepdims=True)
        oheads.append(jnp.dot(p, vh, preferred_element_type=jnp.float32))
    o2 = jnp.concatenate(oheads, axis=1)
    z2 = z2 + jnp.dot(o2.astype(jnp.bfloat16), wo_ref[...],
                      preferred_element_type=jnp.float32)
    zn2 = _ln(z2, ln2g_ref[...], ln2b_ref[...])
    h1 = jax.nn.gelu(
        jnp.dot(zn2.astype(jnp.bfloat16), w1_ref[...],
                preferred_element_type=jnp.float32) + b1_ref[...])
    z2 = z2 + jnp.dot(h1.astype(jnp.bfloat16), w2_ref[...],
                      preferred_element_type=jnp.float32) + b2_ref[...]
    out_ref[0] = z2.reshape(2 * WS, W, D)


def _swin_call(z, p, shift):
    if shift:
        z = jnp.roll(z, (-shift, -shift), (1, 2))
    bf = jnp.bfloat16
    weights = [p['wqkv'].astype(bf), p['wo'].astype(bf),
               p['ln1_g'].reshape(1, D),
               p['ln1_b'].reshape(1, D), p['ln2_g'].reshape(1, D),
               p['ln2_b'].reshape(1, D), p['w1'].astype(bf),
               p['b1'].reshape(1, FF),
               p['w2'].astype(bf), p['b2'].reshape(1, D)]
    out = pl.pallas_call(
        _swin_body,
        grid=(M, NWR // 2),
        in_specs=[pl.BlockSpec((1, 2 * WS, W, D), lambda m, r: (m, r, 0, 0))]
        + [_full_spec(w) for w in weights],
        out_specs=pl.BlockSpec((1, 2 * WS, W, D), lambda m, r: (m, r, 0, 0)),
        out_shape=jax.ShapeDtypeStruct((M, H, W, D), jnp.float32),
        compiler_params=pltpu.CompilerParams(
            dimension_semantics=("parallel", "parallel")),
    )(z, *weights)
    if shift:
        out = jnp.roll(out, (shift, shift), (1, 2))
    return out


def _kvproj_body(zc_ref, wkv_ref, g_ref, b_ref, out_ref):
    kn = _ln(zc_ref[0], g_ref[...], b_ref[...])
    kv = jnp.dot(kn.astype(jnp.bfloat16), wkv_ref[...],
                 preferred_element_type=jnp.float32)
    kb = jax.lax.bitcast_convert_type(
        kv[:, 0:D].astype(jnp.bfloat16), jnp.uint16).astype(jnp.int32)
    vb = jax.lax.bitcast_convert_type(
        kv[:, D:2 * D].astype(jnp.bfloat16), jnp.uint16).astype(jnp.int32)
    out_ref[0] = jax.lax.shift_left(kb, 16) | vb


def _kvproj_call(zc_flat, p):
    weights = [p['wkv'].astype(jnp.bfloat16), p['lnkv_g'].reshape(1, D),
               p['lnkv_b'].reshape(1, D)]
    return pl.pallas_call(
        _kvproj_body,
        grid=(M,),
        in_specs=[pl.BlockSpec((1, HW, D), lambda m: (m, 0, 0))]
        + [_full_spec(w) for w in weights],
        out_specs=pl.BlockSpec((1, HW, D), lambda m: (m, 0, 0)),
        out_shape=jax.ShapeDtypeStruct((M, HW, D), jnp.int32),
        compiler_params=pltpu.CompilerParams(
            dimension_semantics=("parallel",)),
    )(zc_flat, *weights)


_GB = K * M * NT
_CH = 128


@functools.cache
def _make_sc_gather():
    info = plsc.get_sparse_core_info()
    nw = info.num_cores * info.num_subcores
    b_per_w = _GB // nw
    n_chunks = b_per_w // _CH
    mesh = plsc.VectorSubcoreMesh(core_axis_name="c", subcore_axis_name="s")

    @functools.partial(
        pl.kernel, mesh=mesh,
        out_type=jax.ShapeDtypeStruct((_GB, D), jnp.int32),
        scratch_types=[
            pltpu.VMEM((b_per_w,), jnp.int32),
            pltpu.VMEM((2, _CH, D), jnp.int32),
            pltpu.SemaphoreType.DMA,
            pltpu.SemaphoreType.DMA,
        ],
    )
    def sc_gather(table_hbm, idx_hbm, out_hbm, idx_v, rows_v, sem0, sem1):
        wid = lax.axis_index("s") * info.num_cores + lax.axis_index("c")
        base = wid * b_per_w
        pltpu.sync_copy(idx_hbm.at[pl.ds(base, b_per_w)], idx_v)
        sems = (sem0, sem1)
        cps = {}
        cps[0] = pltpu.async_copy(
            table_hbm.at[idx_v.at[pl.ds(0, _CH)]], rows_v.at[0], sems[0])
        for c in range(1, n_chunks):
            b = c % 2
            cps[c] = pltpu.async_copy(
                table_hbm.at[idx_v.at[pl.ds(c * _CH, _CH)]],
                rows_v.at[b], sems[b])
            cps[c - 1].wait()
            pltpu.sync_copy(rows_v.at[1 - b],
                            out_hbm.at[pl.ds(base + (c - 1) * _CH, _CH)])
        last = n_chunks - 1
        cps[last].wait()
        pltpu.sync_copy(rows_v.at[last % 2],
                        out_hbm.at[pl.ds(base + last * _CH, _CH)])

    return sc_gather


def _gather_rows(table, idx_flat):
    return _make_sc_gather()(table, idx_flat)


def _mhca_body(zt_ref, g_ref, wq_ref, wo_ref, lnqg_ref, lnqb_ref, ln2g_ref,
               ln2b_ref, w1_ref, b1_ref, w2_ref, b2_ref, out_ref):
    ztb = zt_ref[0]
    qn = _ln(ztb, lnqg_ref[...], lnqb_ref[...])
    q = jnp.dot(qn.astype(jnp.bfloat16), wq_ref[...],
                preferred_element_type=jnp.float32)
    seg = (jax.lax.broadcasted_iota(jnp.int32, (D, NH), 0) // DH
           == jax.lax.broadcasted_iota(jnp.int32, (D, NH), 1)
           ).astype(jnp.float32)
    segT = (jax.lax.broadcasted_iota(jnp.int32, (NH, D), 0)
            == jax.lax.broadcasted_iota(jnp.int32, (NH, D), 1) // DH
            ).astype(jnp.float32)
    def _unpack_hi(w):
        return jax.lax.bitcast_convert_type(
            jax.lax.shift_right_logical(w, 16).astype(jnp.uint16),
            jnp.bfloat16).astype(jnp.float32)

    def _unpack_lo(w):
        return jax.lax.bitcast_convert_type(
            w.astype(jnp.uint16), jnp.bfloat16).astype(jnp.float32)

    s_list = []
    for j in range(K):
        kj = _unpack_hi(g_ref[j, 0])
        s_list.append(jnp.dot(q * kj, seg,
                              preferred_element_type=jnp.float32) * _SCALE)
    mx = s_list[0]
    for j in range(1, K):
        mx = jnp.maximum(mx, s_list[j])
    e_list = [jnp.exp(s - mx) for s in s_list]
    den = e_list[0]
    for j in range(1, K):
        den = den + e_list[j]
    inv = 1.0 / den
    acc = jnp.zeros((NT, D), jnp.float32)
    for j in range(K):
        vj = _unpack_lo(g_ref[j, 0])
        a_exp = jnp.dot(e_list[j] * inv, segT,
                        preferred_element_type=jnp.float32)
        acc = acc + a_exp * vj
    z = ztb + jnp.dot(acc.astype(jnp.bfloat16), wo_ref[...],
                      preferred_element_type=jnp.float32)
    zn2 = _ln(z, ln2g_ref[...], ln2b_ref[...])
    h1 = jax.nn.gelu(
        jnp.dot(zn2.astype(jnp.bfloat16), w1_ref[...],
                preferred_element_type=jnp.float32) + b1_ref[...])
    z = z + jnp.dot(h1.astype(jnp.bfloat16), w2_ref[...],
                    preferred_element_type=jnp.float32) + b2_ref[...]
    out_ref[0] = z


def _mhca_call(zt, gathered, p):
    g4 = gathered.reshape(K, M, NT, D)
    bf = jnp.bfloat16
    weights = [p['wq'].astype(bf), p['wo'].astype(bf),
               p['lnq_g'].reshape(1, D),
               p['lnq_b'].reshape(1, D), p['ln2_g'].reshape(1, D),
               p['ln2_b'].reshape(1, D), p['w1'].astype(bf),
               p['b1'].reshape(1, FF),
               p['w2'].astype(bf), p['b2'].reshape(1, D)]
    return pl.pallas_call(
        _mhca_body,
        grid=(M,),
        in_specs=[
            pl.BlockSpec((1, NT, D), lambda m: (m, 0, 0)),
            pl.BlockSpec((K, 1, NT, D), lambda m: (0, m, 0, 0)),
        ] + [_full_spec(w) for w in weights],
        out_specs=pl.BlockSpec((1, NT, D), lambda m: (m, 0, 0)),
        out_shape=jax.ShapeDtypeStruct((M, NT, D), jnp.float32),
        compiler_params=pltpu.CompilerParams(
            dimension_semantics=("parallel",)),
    )(zt, g4, *weights)


def kernel(xc, zc, xt, zt, params):
    xc_t = xc.reshape(M, HW, DX).transpose(0, 2, 1)
    idx = _topk_call(xt, xc_t)
    idx_flat = (idx.transpose(2, 0, 1)
                + (HW * jnp.arange(M, dtype=jnp.int32))[None, :, None]
                ).reshape(_GB)
    for layer in params['layers']:
        zc = _swin_call(zc, layer['swin'][0], 0)
        zc = _swin_call(zc, layer['swin'][1], WS // 2)
        kv = _kvproj_call(zc.reshape(M, HW, D), layer['mhca'])
        gathered = _gather_rows(kv.reshape(M * HW, D), idx_flat)
        zt = _mhca_call(zt, gathered, layer['mhca'])
    return zt
